# baseline (speedup 1.0000x reference)
import functools

import jax
import jax.numpy as jnp
from jax import lax
from jax.experimental import pallas as pl
from jax.experimental.pallas import tpu as pltpu
from jax.experimental.pallas import tpu_sc as plsc

N = 10000
D = 128
E = 320000

NC = 2
NS = 16
NW = NC * NS
L = 16
CH1 = 80
CH2 = 128
E_PAD = 327680
N_ACC = 10112
RPT = N_ACC // NS

_mesh = plsc.VectorSubcoreMesh(
    core_axis_name="c", subcore_axis_name="s", num_cores=NC, num_subcores=NS)


def _row_chunks(ch):
    out = []
    off = 0
    while off < RPT:
        sz = min(ch, RPT - off)
        out.append((off, sz))
        off += sz
    return out


def _sc_agg_body(with_cnt, ch, nchunk, *refs):
    if with_cnt:
        (table, src, dst, zrow, zhist,
         parts, chist,
         sidx0, sidx1, didx0, didx1, rows0, rows1, hist,
         acc, sem0, sem1) = refs
    else:
        (table, src, dst, zrow,
         parts,
         sidx0, sidx1, didx0, didx1, rows0, rows1,
         acc, sem0, sem1) = refs
    sidx = (sidx0, sidx1)
    didx = (didx0, didx1)
    rows = (rows0, rows1)
    sems = (sem0, sem1)

    c = lax.axis_index("c")
    s = lax.axis_index("s")
    w = s * NC + c
    epw = nchunk * ch

    pltpu.sync_copy(zrow, rows0)
    for off, sz in _row_chunks(ch):
        pltpu.sync_copy(rows0.at[pl.ds(0, sz)],
                        acc.at[pl.ds(s * RPT + off, sz)])
    if with_cnt:
        pltpu.sync_copy(zhist, hist)
    plsc.subcore_barrier()

    ones16 = jnp.ones((L,), jnp.float32)

    def load_and_gather(b, j):
        base = pl.multiple_of(w * epw + j * ch, ch)
        pltpu.sync_copy(src.at[pl.ds(base, ch)], sidx[b])
        pltpu.sync_copy(dst.at[pl.ds(base, ch)], didx[b])
        pltpu.async_copy(table.at[sidx[b]], rows[b], sems[b])

    def consume(b):
        pltpu.make_async_copy(table.at[sidx[b]], rows[b], sems[b]).wait()
        pltpu.sync_copy(rows[b], acc.at[didx[b]], add=True)
        if with_cnt:
            for jj in range(ch // L):
                idx16 = didx[b][pl.ds(jj * L, L)]
                plsc.addupdate_scatter(hist, [idx16], ones16)

    load_and_gather(0, 0)
    load_and_gather(1, 1)

    def pair(g, carry):
        for b in range(2):
            consume(b)
            load_and_gather(b, 2 * g + b + 2)
        return carry

    lax.fori_loop(0, nchunk // 2 - 1, pair, 0)
    consume(0)
    consume(1)
    plsc.subcore_barrier()

    for off, sz in _row_chunks(ch):
        pltpu.sync_copy(acc.at[pl.ds(s * RPT + off, sz)],
                        rows0.at[pl.ds(0, sz)])
        pltpu.sync_copy(rows0.at[pl.ds(0, sz)],
                        parts.at[c, pl.ds(s * RPT + off, sz)])
    if with_cnt:
        pltpu.sync_copy(hist, chist.at[w])


def _sc_agg(table, src, dst, with_cnt):
    ch = CH1 if with_cnt else CH2
    nchunk = E_PAD // NW // ch
    zrow = jnp.zeros((ch, D), jnp.float32)
    out_type = [jax.ShapeDtypeStruct((NC, N_ACC, D), jnp.float32)]
    scratch = [
        pltpu.VMEM((ch,), jnp.int32),
        pltpu.VMEM((ch,), jnp.int32),
        pltpu.VMEM((ch,), jnp.int32),
        pltpu.VMEM((ch,), jnp.int32),
        pltpu.VMEM((ch, D), jnp.float32),
        pltpu.VMEM((ch, D), jnp.float32),
    ]
    if with_cnt:
        out_type.append(jax.ShapeDtypeStruct((NW, N_ACC), jnp.float32))
        scratch.append(pltpu.VMEM((N_ACC,), jnp.float32))
    scratch.append(pltpu.VMEM_SHARED((N_ACC, D), jnp.float32))
    scratch.append(pltpu.SemaphoreType.DMA)
    scratch.append(pltpu.SemaphoreType.DMA)

    kern = pl.kernel(
        functools.partial(_sc_agg_body, with_cnt, ch, nchunk),
        out_type=out_type,
        mesh=_mesh,
        scratch_types=scratch,
        compiler_params=pltpu.CompilerParams(needs_layout_passes=False),
    )
    if with_cnt:
        zhist = jnp.zeros((N_ACC,), jnp.float32)
        return kern(table, src, dst, zrow, zhist)
    return kern(table, src, dst, zrow)[0]


def _tc1_body(p0, p1, ch, x, wl, b, wr, out, inv_out):
    agg = p0[0] + p1[0]
    cnt = jnp.sum(ch[...], axis=1)[:, None]
    inv = 1.0 / jnp.maximum(cnt, 1.0)
    mean = agg * inv
    y = (jnp.dot(mean, wl[...], preferred_element_type=jnp.float32)
         + jnp.dot(x[...], wr[...], preferred_element_type=jnp.float32)
         + b[...])
    out[...] = jnp.maximum(y, 0.0)
    inv_out[...] = jnp.broadcast_to(inv, inv_out.shape)


def _tc2_body(p0, p1, inv_in, x, wl, b, wr, out):
    agg = p0[0] + p1[0]
    inv = inv_in[:, :1]
    mean = agg * inv
    out[...] = (jnp.dot(mean, wl[...], preferred_element_type=jnp.float32)
                + jnp.dot(x[...], wr[...], preferred_element_type=jnp.float32)
                + b[...])


_RB = 632
_NB = N_ACC // _RB


def _tc_layer1(parts, chist, x, wl, b, wr):
    return pl.pallas_call(
        _tc1_body,
        grid=(_NB,),
        in_specs=[
            pl.BlockSpec((1, _RB, D), lambda i: (0, i, 0)),
            pl.BlockSpec((1, _RB, D), lambda i: (1, i, 0)),
            pl.BlockSpec((_RB, NW), lambda i: (i, 0)),
            pl.BlockSpec((_RB, D), lambda i: (i, 0)),
            pl.BlockSpec((D, D), lambda i: (0, 0)),
            pl.BlockSpec((1, D), lambda i: (0, 0)),
            pl.BlockSpec((D, D), lambda i: (0, 0)),
        ],
        out_specs=[
            pl.BlockSpec((_RB, D), lambda i: (i, 0)),
            pl.BlockSpec((_RB, 8), lambda i: (i, 0)),
        ],
        out_shape=[
            jax.ShapeDtypeStruct((N_ACC, D), jnp.float32),
            jax.ShapeDtypeStruct((N_ACC, 8), jnp.float32),
        ],
    )(parts, parts, chist, x, wl, b, wr)


def _tc_layer2(parts, inv, x, wl, b, wr):
    return pl.pallas_call(
        _tc2_body,
        grid=(_NB,),
        in_specs=[
            pl.BlockSpec((1, _RB, D), lambda i: (0, i, 0)),
            pl.BlockSpec((1, _RB, D), lambda i: (1, i, 0)),
            pl.BlockSpec((_RB, 8), lambda i: (i, 0)),
            pl.BlockSpec((_RB, D), lambda i: (i, 0)),
            pl.BlockSpec((D, D), lambda i: (0, 0)),
            pl.BlockSpec((1, D), lambda i: (0, 0)),
            pl.BlockSpec((D, D), lambda i: (0, 0)),
        ],
        out_specs=pl.BlockSpec((_RB, D), lambda i: (i, 0)),
        out_shape=jax.ShapeDtypeStruct((N_ACC, D), jnp.float32),
    )(parts, parts, inv, x, wl, b, wr)


@jax.jit
def kernel(x, edge_index, W1_l, b1_l, W1_r, W2_l, b2_l, W2_r):
    src = edge_index[0]
    dst = edge_index[1]
    npad = E_PAD - E
    src_p = jnp.concatenate([src, jnp.zeros((npad,), jnp.int32)])
    dst_p = jnp.concatenate([dst, jnp.full((npad,), N, jnp.int32)])
    xp = jnp.pad(x, ((0, N_ACC - N), (0, 0)))

    parts1, chist = _sc_agg(xp, src_p, dst_p, with_cnt=True)
    h, inv = _tc_layer1(parts1, chist.T, xp, W1_l, b1_l.reshape(1, D),
                        W1_r)
    parts2 = _sc_agg(h, src_p, dst_p, with_cnt=False)
    out = _tc_layer2(parts2, inv, h, W2_l, b2_l.reshape(1, D), W2_r)
    return out[:N]
